# baseline (speedup 1.0000x reference)
import numpy as np
import jax
import jax.numpy as jnp
from jax import lax
from jax.experimental import pallas as pl
from jax.experimental.pallas import tpu as pltpu
from jax.experimental.pallas import tpu_sc as plsc

NC = 2
NS = 16
NW = NC * NS
LANES = 16
CHUNK = 80
ZR = 80
CU = 4


def _bf16_perm(w):
    p = np.empty(w, np.int32)
    for k in range(w // 32):
        for j in range(16):
            p[32 * k + 2 * j] = 16 * k + j
            p[32 * k + 2 * j + 1] = w // 2 + 16 * k + j
    return p


def _pack_bf16(a, w):
    m = a.shape[0]
    b = a[:, _bf16_perm(w)].astype(jnp.bfloat16).reshape(m, w // 2, 2)
    return lax.bitcast_convert_type(b, jnp.int32)


def _zero_vmem_2d(ref, nrow, wv):
    zf = jnp.zeros((LANES,), jnp.float32)

    def zbody(i, _):
        ref[i // wv, pl.ds((i % wv) * LANES, LANES)] = zf
        return 0

    lax.fori_loop(0, nrow * wv, zbody, 0)


def _fill_vmem_1d(ref, nelem, value):
    v = jnp.full((LANES,), value, jnp.float32)

    def fbody(i, _):
        ref[pl.ds(i * LANES, LANES)] = v
        return 0

    lax.fori_loop(0, nelem // LANES, fbody, 0)


def _convert_rows(bfbuf, fbuf, w):
    vecs = w // 32

    def cbody(i, _):
        for u in range(CU):
            r = i * CU + u
            for k in range(vecs):
                v = bfbuf[r, pl.ds(k * 16, 16)]
                fbuf[r, pl.ds(k * 16, 16)] = plsc.bitcast(
                    v << 16, jnp.float32)
                fbuf[r, pl.ds(w // 2 + k * 16, 16)] = plsc.bitcast(
                    v & (-65536), jnp.float32)
        return 0

    lax.fori_loop(0, CHUNK // CU, cbody, 0)


def _sc_layer0(xsrc, ei4, npad, width):
    nchunk1 = ei4.shape[2]
    nchunk = 2 * nchunk1
    rz = npad // NS

    def body(ei_hbm, xsrc_hbm, out_hbm, deg_hbm, src_all, dst_all, bf0, bf1,
             f0buf, f1buf, zrow, ones, zdeg, acc, deg_sh,
             gs0, gs1, ss0, ss1, ds0, ds1):
        c = lax.axis_index("c")
        s = lax.axis_index("s")
        r0 = s * rz

        pltpu.sync_copy(ei_hbm.at[0, 2 * s, :, :],
                        src_all.at[pl.ds(0, nchunk1), :])
        pltpu.sync_copy(ei_hbm.at[0, 2 * s + 1, :, :],
                        src_all.at[pl.ds(nchunk1, nchunk1), :])
        pltpu.sync_copy(ei_hbm.at[1, 2 * s, :, :],
                        dst_all.at[pl.ds(0, nchunk1), :])
        pltpu.sync_copy(ei_hbm.at[1, 2 * s + 1, :, :],
                        dst_all.at[pl.ds(nchunk1, nchunk1), :])

        cw = jnp.full((LANES,), 0, jnp.int32) + c

        def tbody(i, _):
            j = i // (CHUNK // LANES)
            k = i % (CHUNK // LANES)
            v = src_all[j, pl.ds(k * LANES, LANES)]
            src_all[j, pl.ds(k * LANES, LANES)] = v * 2 + cw
            return 0

        lax.fori_loop(0, nchunk * (CHUNK // LANES), tbody, 0)

        _zero_vmem_2d(zrow, ZR, width // LANES)
        _fill_vmem_1d(zdeg, rz, 0.0)
        _fill_vmem_1d(ones, CHUNK, 1.0)

        def zacc_body(i, _):
            pltpu.sync_copy(zrow, acc.at[pl.ds(r0 + i * ZR, ZR), :])
            return 0

        lax.fori_loop(0, rz // ZR, zacc_body, 0)
        pltpu.sync_copy(zdeg, deg_sh.at[pl.ds(r0, rz)])
        plsc.subcore_barrier()

        def g_start(j, buf, sem):
            pltpu.async_copy(xsrc_hbm.at[src_all.at[j]], buf, sem)

        def g_wait(buf, sem):
            pltpu.make_async_copy(xsrc_hbm.at[src_all.at[0]], buf, sem).wait()

        def s_start(j, fbuf, ssem, dsem):
            drow = dst_all.at[j]
            pltpu.async_copy(fbuf, acc.at[drow], ssem, add=True)
            pltpu.async_copy(ones, deg_sh.at[drow], dsem, add=True)

        def s_wait(fbuf, ssem, dsem):
            pltpu.make_async_copy(fbuf, acc.at[dst_all.at[0]], ssem).wait()
            pltpu.make_async_copy(ones, deg_sh.at[dst_all.at[0]], dsem).wait()

        g_start(0, bf0, gs0)
        g_start(1, bf1, gs1)

        def pair_body(i, _):
            j0 = 2 * i
            g_wait(bf0, gs0)

            @pl.when(i > 0)
            def _():
                s_wait(f0buf, ss0, ds0)

            _convert_rows(bf0, f0buf, width)
            s_start(j0, f0buf, ss0, ds0)

            @pl.when(j0 + 2 < nchunk)
            def _():
                g_start(j0 + 2, bf0, gs0)

            g_wait(bf1, gs1)

            @pl.when(i > 0)
            def _():
                s_wait(f1buf, ss1, ds1)

            _convert_rows(bf1, f1buf, width)
            s_start(j0 + 1, f1buf, ss1, ds1)

            @pl.when(j0 + 3 < nchunk)
            def _():
                g_start(j0 + 3, bf1, gs1)

            return 0

        lax.fori_loop(0, nchunk // 2, pair_body, 0)
        if nchunk % 2:
            g_wait(bf0, gs0)
            s_wait(f0buf, ss0, ds0)
            _convert_rows(bf0, f0buf, width)
            s_start(nchunk - 1, f0buf, ss0, ds0)
        s_wait(f0buf, ss0, ds0)
        s_wait(f1buf, ss1, ds1)
        plsc.subcore_barrier()

        pltpu.sync_copy(acc.at[pl.ds(r0, rz), :],
                        out_hbm.at[c, pl.ds(r0, rz), :])
        pltpu.sync_copy(deg_sh.at[pl.ds(r0, rz)],
                        deg_hbm.at[c, pl.ds(r0, rz)])

    mesh = plsc.VectorSubcoreMesh(core_axis_name="c", subcore_axis_name="s")
    kfn = pl.kernel(
        body,
        mesh=mesh,
        out_type=[
            jax.ShapeDtypeStruct((NC, npad, width), jnp.float32),
            jax.ShapeDtypeStruct((NC, npad), jnp.float32),
        ],
        scratch_types=[
            pltpu.VMEM((nchunk, CHUNK), jnp.int32),
            pltpu.VMEM((nchunk, CHUNK), jnp.int32),
            pltpu.VMEM((CHUNK, width // 2), jnp.int32),
            pltpu.VMEM((CHUNK, width // 2), jnp.int32),
            pltpu.VMEM((CHUNK, width), jnp.float32),
            pltpu.VMEM((CHUNK, width), jnp.float32),
            pltpu.VMEM((ZR, width), jnp.float32),
            pltpu.VMEM((CHUNK,), jnp.float32),
            pltpu.VMEM((rz,), jnp.float32),
            pltpu.VMEM_SHARED((npad, width), jnp.float32),
            pltpu.VMEM_SHARED((npad,), jnp.float32),
            pltpu.SemaphoreType.DMA,
            pltpu.SemaphoreType.DMA,
            pltpu.SemaphoreType.DMA,
            pltpu.SemaphoreType.DMA,
            pltpu.SemaphoreType.DMA,
            pltpu.SemaphoreType.DMA,
        ],
        compiler_params=pltpu.CompilerParams(use_tc_tiling_on_sc=False, needs_layout_passes=False),
    )
    return kfn(ei4, xsrc)


def _sc_layer1(gsrc, ei4, npad, width):
    nchunk = ei4.shape[2]
    rz = npad // NS

    def body(ei_hbm, g_hbm, out_hbm, src_all, dst_all, bf0, bf1,
             f0buf, f1buf, zrow, acc, gs0, gs1, ss0, ss1):
        c = lax.axis_index("c")
        s = lax.axis_index("s")
        wid = c * NS + s
        r0 = s * rz

        pltpu.sync_copy(ei_hbm.at[0, wid, :, :], src_all)
        pltpu.sync_copy(ei_hbm.at[1, wid, :, :], dst_all)

        _zero_vmem_2d(zrow, ZR, width // LANES)

        def zacc_body(i, _):
            pltpu.sync_copy(zrow, acc.at[pl.ds(r0 + i * ZR, ZR), :])
            return 0

        lax.fori_loop(0, rz // ZR, zacc_body, 0)
        plsc.subcore_barrier()

        def g_start(j, buf, sem):
            pltpu.async_copy(g_hbm.at[src_all.at[j]], buf, sem)

        def g_wait(buf, sem):
            pltpu.make_async_copy(g_hbm.at[src_all.at[0]], buf, sem).wait()

        def s_start(j, fbuf, ssem):
            pltpu.async_copy(fbuf, acc.at[dst_all.at[j]], ssem, add=True)

        def s_wait(fbuf, ssem):
            pltpu.make_async_copy(fbuf, acc.at[dst_all.at[0]], ssem).wait()

        g_start(0, bf0, gs0)
        g_start(1, bf1, gs1)

        def pair_body(i, _):
            j0 = 2 * i
            g_wait(bf0, gs0)

            @pl.when(i > 0)
            def _():
                s_wait(f0buf, ss0)

            _convert_rows(bf0, f0buf, width)
            s_start(j0, f0buf, ss0)

            @pl.when(j0 + 2 < nchunk)
            def _():
                g_start(j0 + 2, bf0, gs0)

            g_wait(bf1, gs1)

            @pl.when(i > 0)
            def _():
                s_wait(f1buf, ss1)

            _convert_rows(bf1, f1buf, width)
            s_start(j0 + 1, f1buf, ss1)

            @pl.when(j0 + 3 < nchunk)
            def _():
                g_start(j0 + 3, bf1, gs1)

            return 0

        lax.fori_loop(0, nchunk // 2, pair_body, 0)
        if nchunk % 2:
            g_wait(bf0, gs0)
            s_wait(f0buf, ss0)
            _convert_rows(bf0, f0buf, width)
            s_start(nchunk - 1, f0buf, ss0)
        s_wait(f0buf, ss0)
        s_wait(f1buf, ss1)
        plsc.subcore_barrier()

        pltpu.sync_copy(acc.at[pl.ds(r0, rz), :],
                        out_hbm.at[c, pl.ds(r0, rz), :])

    mesh = plsc.VectorSubcoreMesh(core_axis_name="c", subcore_axis_name="s")
    kfn = pl.kernel(
        body,
        mesh=mesh,
        out_type=[jax.ShapeDtypeStruct((NC, npad, width), jnp.float32)],
        scratch_types=[
            pltpu.VMEM((nchunk, CHUNK), jnp.int32),
            pltpu.VMEM((nchunk, CHUNK), jnp.int32),
            pltpu.VMEM((CHUNK, width // 2), jnp.int32),
            pltpu.VMEM((CHUNK, width // 2), jnp.int32),
            pltpu.VMEM((CHUNK, width), jnp.float32),
            pltpu.VMEM((CHUNK, width), jnp.float32),
            pltpu.VMEM((ZR, width), jnp.float32),
            pltpu.VMEM_SHARED((npad, width), jnp.float32),
            pltpu.SemaphoreType.DMA,
            pltpu.SemaphoreType.DMA,
            pltpu.SemaphoreType.DMA,
            pltpu.SemaphoreType.DMA,
        ],
        compiler_params=pltpu.CompilerParams(use_tc_tiling_on_sc=False, needs_layout_passes=False),
    )
    return kfn(ei4, gsrc)[0]


def _tc_layer0(p, xh, deg3, W0L, W0R, W1):
    n = xh.shape[0]
    half = xh.shape[2]
    hid = W0L.shape[0]
    nz = W1.shape[0]
    R = 2000

    def body(p_ref, x_ref, d_ref, w0l_ref, w0r_ref, w1_ref, o_ref):
        inv = 1.0 / (d_ref[...] + 1.0)
        rstL = (p_ref[0] + x_ref[:, 0, :]) * inv
        rstR = (p_ref[1] + x_ref[:, 1, :]) * inv
        h = lax.dot_general(rstL, w0l_ref[...], (((1,), (1,)), ((), ())),
                            preferred_element_type=jnp.float32)
        h = h + lax.dot_general(rstR, w0r_ref[...], (((1,), (1,)), ((), ())),
                                preferred_element_type=jnp.float32)
        h = jnp.maximum(h, 0.0)
        o_ref[...] = lax.dot_general(h, w1_ref[...], (((1,), (1,)), ((), ())),
                                     preferred_element_type=jnp.float32)

    return pl.pallas_call(
        body,
        grid=(n // R,),
        in_specs=[
            pl.BlockSpec((2, R, half), lambda i: (0, i, 0)),
            pl.BlockSpec((R, 2, half), lambda i: (i, 0, 0)),
            pl.BlockSpec((R, 1), lambda i: (i, 0)),
            pl.BlockSpec((hid, half), lambda i: (0, 0)),
            pl.BlockSpec((hid, half), lambda i: (0, 0)),
            pl.BlockSpec((nz, hid), lambda i: (0, 0)),
        ],
        out_specs=pl.BlockSpec((R, nz), lambda i: (i, 0)),
        out_shape=jax.ShapeDtypeStruct((n, nz), jnp.float32),
    )(p, xh, deg3, W0L, W0R, W1)


def _tc_layer1(q, g, deg3):
    n, nz = g.shape

    R = 2000

    def body(q_ref, g_ref, d_ref, o_ref):
        inv = 1.0 / (d_ref[...] + 1.0)
        o_ref[...] = jnp.maximum((q_ref[0] + q_ref[1] + g_ref[...]) * inv, 0.0)

    return pl.pallas_call(
        body,
        grid=(n // R,),
        in_specs=[
            pl.BlockSpec((2, R, nz), lambda i: (0, i, 0)),
            pl.BlockSpec((R, nz), lambda i: (i, 0)),
            pl.BlockSpec((R, 1), lambda i: (i, 0)),
        ],
        out_specs=pl.BlockSpec((R, nz), lambda i: (i, 0)),
        out_shape=jax.ShapeDtypeStruct((n, nz), jnp.float32),
    )(q, g, deg3)


def kernel(x, edge_index, W0, W1):
    n, d_in = x.shape
    half = d_in // 2
    e = edge_index.shape[1]
    npad = ((n + 1023) // 1024) * 1024

    ei4 = edge_index.astype(jnp.int32).reshape(2, NW, e // (NW * CHUNK), CHUNK)

    x2 = x.reshape(2 * n, half)
    xh = x.reshape(n, 2, half)
    xsrc = _pack_bf16(x2, half)

    p, deg2 = _sc_layer0(xsrc, ei4, npad, half)
    deg3 = deg2[0, :n, None]

    g = _tc_layer0(p, xh, deg3, W0[:, :half], W0[:, half:], W1)

    gsrc = _pack_bf16(g, half)
    q = _sc_layer1(gsrc, ei4, npad, half)

    return _tc_layer1(q, g, deg3)


# speedup vs baseline: 1.4106x; 1.4106x over previous
import jax
import jax.numpy as jnp
from jax import lax
from jax.experimental import pallas as pl
from jax.experimental.pallas import tpu as pltpu
from jax.experimental.pallas import tpu_sc as plsc

NC = 2
NS = 16
NW = NC * NS
LANES = 16
CHUNK = 80
ZR = 80


def _zero_vmem_2d(ref, nrow, wv):
    zf = jnp.zeros((LANES,), jnp.float32)

    def zbody(i, _):
        ref[i // wv, pl.ds((i % wv) * LANES, LANES)] = zf
        return 0

    lax.fori_loop(0, nrow * wv, zbody, 0)


def _fill_vmem_1d(ref, nelem, value):
    v = jnp.full((LANES,), value, jnp.float32)

    def fbody(i, _):
        ref[pl.ds(i * LANES, LANES)] = v
        return 0

    lax.fori_loop(0, nelem // LANES, fbody, 0)


def _sc_layer0(x2, ei4, npad, width):
    nchunk1 = ei4.shape[2]
    nchunk = 2 * nchunk1
    rz = npad // NS

    def body(ei_hbm, x2_hbm, out_hbm, deg_hbm, src_all, dst_all, rows0, rows1,
             zrow, ones, zdeg, acc, deg_sh, gs0, gs1):
        c = lax.axis_index("c")
        s = lax.axis_index("s")
        r0 = s * rz

        pltpu.sync_copy(ei_hbm.at[0, 2 * s, :, :],
                        src_all.at[pl.ds(0, nchunk1), :])
        pltpu.sync_copy(ei_hbm.at[0, 2 * s + 1, :, :],
                        src_all.at[pl.ds(nchunk1, nchunk1), :])
        pltpu.sync_copy(ei_hbm.at[1, 2 * s, :, :],
                        dst_all.at[pl.ds(0, nchunk1), :])
        pltpu.sync_copy(ei_hbm.at[1, 2 * s + 1, :, :],
                        dst_all.at[pl.ds(nchunk1, nchunk1), :])

        cw = jnp.full((LANES,), 0, jnp.int32) + c

        def tbody(i, _):
            j = i // (CHUNK // LANES)
            k = i % (CHUNK // LANES)
            v = src_all[j, pl.ds(k * LANES, LANES)]
            src_all[j, pl.ds(k * LANES, LANES)] = v * 2 + cw
            return 0

        lax.fori_loop(0, nchunk * (CHUNK // LANES), tbody, 0)

        _zero_vmem_2d(zrow, ZR, width // LANES)
        _fill_vmem_1d(zdeg, rz, 0.0)
        _fill_vmem_1d(ones, CHUNK, 1.0)

        def zacc_body(i, _):
            pltpu.sync_copy(zrow, acc.at[pl.ds(r0 + i * ZR, ZR), :])
            return 0

        lax.fori_loop(0, rz // ZR, zacc_body, 0)
        pltpu.sync_copy(zdeg, deg_sh.at[pl.ds(r0, rz)])
        plsc.subcore_barrier()

        def g_start(j, buf, sem):
            pltpu.async_copy(x2_hbm.at[src_all.at[j]], buf, sem)

        def g_wait(buf, sem):
            pltpu.make_async_copy(x2_hbm.at[src_all.at[0]], buf, sem).wait()

        def do_scatter(j, buf):
            drow = dst_all.at[j]
            pltpu.sync_copy(buf, acc.at[drow], add=True)
            pltpu.sync_copy(ones, deg_sh.at[drow], add=True)

        g_start(0, rows0, gs0)
        g_start(1, rows1, gs1)

        def pair_body(i, _):
            j0 = 2 * i
            g_wait(rows0, gs0)
            do_scatter(j0, rows0)

            @pl.when(j0 + 2 < nchunk)
            def _():
                g_start(j0 + 2, rows0, gs0)

            g_wait(rows1, gs1)
            do_scatter(j0 + 1, rows1)

            @pl.when(j0 + 3 < nchunk)
            def _():
                g_start(j0 + 3, rows1, gs1)

            return 0

        lax.fori_loop(0, nchunk // 2, pair_body, 0)
        if nchunk % 2:
            g_wait(rows0, gs0)
            do_scatter(nchunk - 1, rows0)
        plsc.subcore_barrier()

        pltpu.sync_copy(acc.at[pl.ds(r0, rz), :],
                        out_hbm.at[c, pl.ds(r0, rz), :])
        pltpu.sync_copy(deg_sh.at[pl.ds(r0, rz)],
                        deg_hbm.at[c, pl.ds(r0, rz)])

    mesh = plsc.VectorSubcoreMesh(core_axis_name="c", subcore_axis_name="s")
    kfn = pl.kernel(
        body,
        mesh=mesh,
        out_type=[
            jax.ShapeDtypeStruct((NC, npad, width), jnp.float32),
            jax.ShapeDtypeStruct((NC, npad), jnp.float32),
        ],
        scratch_types=[
            pltpu.VMEM((nchunk, CHUNK), jnp.int32),
            pltpu.VMEM((nchunk, CHUNK), jnp.int32),
            pltpu.VMEM((CHUNK, width), jnp.float32),
            pltpu.VMEM((CHUNK, width), jnp.float32),
            pltpu.VMEM((ZR, width), jnp.float32),
            pltpu.VMEM((CHUNK,), jnp.float32),
            pltpu.VMEM((rz,), jnp.float32),
            pltpu.VMEM_SHARED((npad, width), jnp.float32),
            pltpu.VMEM_SHARED((npad,), jnp.float32),
            pltpu.SemaphoreType.DMA,
            pltpu.SemaphoreType.DMA,
        ],
        compiler_params=pltpu.CompilerParams(use_tc_tiling_on_sc=False),
    )
    return kfn(ei4, x2)


def _sc_layer1(g, ei4, npad):
    n, width = g.shape
    nchunk = ei4.shape[2]
    rz = npad // NS

    def body(ei_hbm, g_hbm, out_hbm, src_all, dst_all, rows0, rows1, zrow,
             acc, gs0, gs1):
        c = lax.axis_index("c")
        s = lax.axis_index("s")
        wid = c * NS + s
        r0 = s * rz

        pltpu.sync_copy(ei_hbm.at[0, wid, :, :], src_all)
        pltpu.sync_copy(ei_hbm.at[1, wid, :, :], dst_all)

        _zero_vmem_2d(zrow, ZR, width // LANES)

        def zacc_body(i, _):
            pltpu.sync_copy(zrow, acc.at[pl.ds(r0 + i * ZR, ZR), :])
            return 0

        lax.fori_loop(0, rz // ZR, zacc_body, 0)
        plsc.subcore_barrier()

        def g_start(j, buf, sem):
            pltpu.async_copy(g_hbm.at[src_all.at[j]], buf, sem)

        def g_wait(buf, sem):
            pltpu.make_async_copy(g_hbm.at[src_all.at[0]], buf, sem).wait()

        def do_scatter(j, buf):
            pltpu.sync_copy(buf, acc.at[dst_all.at[j]], add=True)

        g_start(0, rows0, gs0)
        g_start(1, rows1, gs1)

        def pair_body(i, _):
            j0 = 2 * i
            g_wait(rows0, gs0)
            do_scatter(j0, rows0)

            @pl.when(j0 + 2 < nchunk)
            def _():
                g_start(j0 + 2, rows0, gs0)

            g_wait(rows1, gs1)
            do_scatter(j0 + 1, rows1)

            @pl.when(j0 + 3 < nchunk)
            def _():
                g_start(j0 + 3, rows1, gs1)

            return 0

        lax.fori_loop(0, nchunk // 2, pair_body, 0)
        if nchunk % 2:
            g_wait(rows0, gs0)
            do_scatter(nchunk - 1, rows0)
        plsc.subcore_barrier()

        pltpu.sync_copy(acc.at[pl.ds(r0, rz), :],
                        out_hbm.at[c, pl.ds(r0, rz), :])

    mesh = plsc.VectorSubcoreMesh(core_axis_name="c", subcore_axis_name="s")
    kfn = pl.kernel(
        body,
        mesh=mesh,
        out_type=[jax.ShapeDtypeStruct((NC, npad, width), jnp.float32)],
        scratch_types=[
            pltpu.VMEM((nchunk, CHUNK), jnp.int32),
            pltpu.VMEM((nchunk, CHUNK), jnp.int32),
            pltpu.VMEM((CHUNK, width), jnp.float32),
            pltpu.VMEM((CHUNK, width), jnp.float32),
            pltpu.VMEM((ZR, width), jnp.float32),
            pltpu.VMEM_SHARED((npad, width), jnp.float32),
            pltpu.SemaphoreType.DMA,
            pltpu.SemaphoreType.DMA,
        ],
        compiler_params=pltpu.CompilerParams(use_tc_tiling_on_sc=False),
    )
    return kfn(ei4, g)[0]


def _tc_layer0(p, xh, deg3, W0L, W0R, W1):
    n = xh.shape[0]
    half = xh.shape[2]
    hid = W0L.shape[0]
    nz = W1.shape[0]
    R = 2000

    def body(p_ref, x_ref, d_ref, w0l_ref, w0r_ref, w1_ref, o_ref):
        inv = 1.0 / (d_ref[...] + 1.0)
        rstL = (p_ref[0] + x_ref[:, 0, :]) * inv
        rstR = (p_ref[1] + x_ref[:, 1, :]) * inv
        h = lax.dot_general(rstL, w0l_ref[...], (((1,), (1,)), ((), ())),
                            preferred_element_type=jnp.float32)
        h = h + lax.dot_general(rstR, w0r_ref[...], (((1,), (1,)), ((), ())),
                                preferred_element_type=jnp.float32)
        h = jnp.maximum(h, 0.0)
        o_ref[...] = lax.dot_general(h, w1_ref[...], (((1,), (1,)), ((), ())),
                                     preferred_element_type=jnp.float32)

    return pl.pallas_call(
        body,
        grid=(n // R,),
        in_specs=[
            pl.BlockSpec((2, R, half), lambda i: (0, i, 0)),
            pl.BlockSpec((R, 2, half), lambda i: (i, 0, 0)),
            pl.BlockSpec((R, 1), lambda i: (i, 0)),
            pl.BlockSpec((hid, half), lambda i: (0, 0)),
            pl.BlockSpec((hid, half), lambda i: (0, 0)),
            pl.BlockSpec((nz, hid), lambda i: (0, 0)),
        ],
        out_specs=pl.BlockSpec((R, nz), lambda i: (i, 0)),
        out_shape=jax.ShapeDtypeStruct((n, nz), jnp.float32),
    )(p, xh, deg3, W0L, W0R, W1)


def _tc_layer1(q, g, deg3):
    n, nz = g.shape

    R = 2000

    def body(q_ref, g_ref, d_ref, o_ref):
        inv = 1.0 / (d_ref[...] + 1.0)
        o_ref[...] = jnp.maximum((q_ref[0] + q_ref[1] + g_ref[...]) * inv, 0.0)

    return pl.pallas_call(
        body,
        grid=(n // R,),
        in_specs=[
            pl.BlockSpec((2, R, nz), lambda i: (0, i, 0)),
            pl.BlockSpec((R, nz), lambda i: (i, 0)),
            pl.BlockSpec((R, 1), lambda i: (i, 0)),
        ],
        out_specs=pl.BlockSpec((R, nz), lambda i: (i, 0)),
        out_shape=jax.ShapeDtypeStruct((n, nz), jnp.float32),
    )(q, g, deg3)


def kernel(x, edge_index, W0, W1):
    n, d_in = x.shape
    half = d_in // 2
    e = edge_index.shape[1]
    npad = ((n + 1023) // 1024) * 1024

    ei4 = edge_index.astype(jnp.int32).reshape(2, NW, e // (NW * CHUNK), CHUNK)

    x2 = x.reshape(2 * n, half)
    xh = x.reshape(n, 2, half)

    p, deg2 = _sc_layer0(x2, ei4, npad, half)
    deg3 = deg2[0, :n, None]

    g = _tc_layer0(p, xh, deg3, W0[:, :half], W0[:, half:], W1)

    q = _sc_layer1(g, ei4, npad)

    return _tc_layer1(q, g, deg3)


# speedup vs baseline: 1.5164x; 1.0750x over previous
import jax
import jax.numpy as jnp
from jax import lax
from jax.experimental import pallas as pl
from jax.experimental.pallas import tpu as pltpu
from jax.experimental.pallas import tpu_sc as plsc

NC = 2
NS = 16
NW = NC * NS
LANES = 16
CHUNK = 80
ZR = 80


def _zero_vmem_2d(ref, nrow, wv):
    zf = jnp.zeros((LANES,), jnp.float32)

    def zbody(i, _):
        ref[i // wv, pl.ds((i % wv) * LANES, LANES)] = zf
        return 0

    lax.fori_loop(0, nrow * wv, zbody, 0)


def _fill_vmem_1d(ref, nelem, value):
    v = jnp.full((LANES,), value, jnp.float32)

    def fbody(i, _):
        ref[pl.ds(i * LANES, LANES)] = v
        return 0

    lax.fori_loop(0, nelem // LANES, fbody, 0)


def _sc_layer0(x2, ei4, npad, width):
    nchunk1 = ei4.shape[2]
    nchunk = 2 * nchunk1
    rz = npad // NS

    def body(ei_hbm, x2_hbm, out_hbm, deg_hbm, src_all, dst_all, rows0, rows1,
             zrow, ones, zdeg, acc, deg_sh, gs0, gs1):
        c = lax.axis_index("c")
        s = lax.axis_index("s")
        r0 = s * rz

        pltpu.sync_copy(ei_hbm.at[0, 2 * s, :, :],
                        src_all.at[pl.ds(0, nchunk1), :])
        pltpu.sync_copy(ei_hbm.at[0, 2 * s + 1, :, :],
                        src_all.at[pl.ds(nchunk1, nchunk1), :])
        pltpu.sync_copy(ei_hbm.at[1, 2 * s, :, :],
                        dst_all.at[pl.ds(0, nchunk1), :])
        pltpu.sync_copy(ei_hbm.at[1, 2 * s + 1, :, :],
                        dst_all.at[pl.ds(nchunk1, nchunk1), :])

        cw = jnp.full((LANES,), 0, jnp.int32) + c

        def tbody(i, _):
            j = i // (CHUNK // LANES)
            k = i % (CHUNK // LANES)
            v = src_all[j, pl.ds(k * LANES, LANES)]
            src_all[j, pl.ds(k * LANES, LANES)] = v * 2 + cw
            return 0

        lax.fori_loop(0, nchunk * (CHUNK // LANES), tbody, 0)

        _zero_vmem_2d(zrow, ZR, width // LANES)
        _fill_vmem_1d(zdeg, rz, 0.0)
        _fill_vmem_1d(ones, CHUNK, 1.0)

        def zacc_body(i, _):
            pltpu.sync_copy(zrow, acc.at[pl.ds(r0 + i * ZR, ZR), :])
            return 0

        lax.fori_loop(0, rz // ZR, zacc_body, 0)
        pltpu.sync_copy(zdeg, deg_sh.at[pl.ds(r0, rz)])
        plsc.subcore_barrier()

        def g_start(j, buf, sem):
            pltpu.async_copy(x2_hbm.at[src_all.at[j]], buf, sem)

        def g_wait(buf, sem):
            pltpu.make_async_copy(x2_hbm.at[src_all.at[0]], buf, sem).wait()

        def do_scatter(j, buf):
            drow = dst_all.at[j]
            pltpu.sync_copy(buf, acc.at[drow], add=True)
            pltpu.sync_copy(ones, deg_sh.at[drow], add=True)

        g_start(0, rows0, gs0)
        g_start(1, rows1, gs1)

        def pair_body(i, _):
            j0 = 2 * i
            g_wait(rows0, gs0)
            do_scatter(j0, rows0)

            @pl.when(j0 + 2 < nchunk)
            def _():
                g_start(j0 + 2, rows0, gs0)

            g_wait(rows1, gs1)
            do_scatter(j0 + 1, rows1)

            @pl.when(j0 + 3 < nchunk)
            def _():
                g_start(j0 + 3, rows1, gs1)

            return 0

        lax.fori_loop(0, nchunk // 2, pair_body, 0)
        if nchunk % 2:
            g_wait(rows0, gs0)
            do_scatter(nchunk - 1, rows0)
        plsc.subcore_barrier()

        pltpu.sync_copy(acc.at[pl.ds(r0, rz), :],
                        out_hbm.at[c, pl.ds(r0, rz), :])
        pltpu.sync_copy(deg_sh.at[pl.ds(r0, rz)],
                        deg_hbm.at[c, pl.ds(r0, rz)])

    mesh = plsc.VectorSubcoreMesh(core_axis_name="c", subcore_axis_name="s")
    kfn = pl.kernel(
        body,
        mesh=mesh,
        out_type=[
            jax.ShapeDtypeStruct((NC, npad, width), jnp.float32),
            jax.ShapeDtypeStruct((NC, npad), jnp.float32),
        ],
        scratch_types=[
            pltpu.VMEM((nchunk, CHUNK), jnp.int32),
            pltpu.VMEM((nchunk, CHUNK), jnp.int32),
            pltpu.VMEM((CHUNK, width), jnp.float32),
            pltpu.VMEM((CHUNK, width), jnp.float32),
            pltpu.VMEM((ZR, width), jnp.float32),
            pltpu.VMEM((CHUNK,), jnp.float32),
            pltpu.VMEM((rz,), jnp.float32),
            pltpu.VMEM_SHARED((npad, width), jnp.float32),
            pltpu.VMEM_SHARED((npad,), jnp.float32),
            pltpu.SemaphoreType.DMA,
            pltpu.SemaphoreType.DMA,
        ],
        compiler_params=pltpu.CompilerParams(use_tc_tiling_on_sc=False),
    )
    return kfn(ei4, x2)


def _sc_layer1(g, ei4, npad):
    n, width = g.shape
    nchunk = ei4.shape[2]
    rz = npad // NS

    def body(ei_hbm, g_hbm, out_hbm, src_all, dst_all, rows0, rows1, zrow,
             acc, gs0, gs1):
        c = lax.axis_index("c")
        s = lax.axis_index("s")
        wid = c * NS + s
        r0 = s * rz

        pltpu.sync_copy(ei_hbm.at[0, wid, :, :], src_all)
        pltpu.sync_copy(ei_hbm.at[1, wid, :, :], dst_all)

        _zero_vmem_2d(zrow, ZR, width // LANES)

        def zacc_body(i, _):
            pltpu.sync_copy(zrow, acc.at[pl.ds(r0 + i * ZR, ZR), :])
            return 0

        lax.fori_loop(0, rz // ZR, zacc_body, 0)
        plsc.subcore_barrier()

        def g_start(j, buf, sem):
            pltpu.async_copy(g_hbm.at[src_all.at[j]], buf, sem)

        def g_wait(buf, sem):
            pltpu.make_async_copy(g_hbm.at[src_all.at[0]], buf, sem).wait()

        def do_scatter(j, buf):
            pltpu.sync_copy(buf, acc.at[dst_all.at[j]], add=True)

        g_start(0, rows0, gs0)
        g_start(1, rows1, gs1)

        def pair_body(i, _):
            j0 = 2 * i
            g_wait(rows0, gs0)
            do_scatter(j0, rows0)

            @pl.when(j0 + 2 < nchunk)
            def _():
                g_start(j0 + 2, rows0, gs0)

            g_wait(rows1, gs1)
            do_scatter(j0 + 1, rows1)

            @pl.when(j0 + 3 < nchunk)
            def _():
                g_start(j0 + 3, rows1, gs1)

            return 0

        lax.fori_loop(0, nchunk // 2, pair_body, 0)
        if nchunk % 2:
            g_wait(rows0, gs0)
            do_scatter(nchunk - 1, rows0)
        plsc.subcore_barrier()

        pltpu.sync_copy(acc.at[pl.ds(r0, rz), :],
                        out_hbm.at[c, pl.ds(r0, rz), :])

    mesh = plsc.VectorSubcoreMesh(core_axis_name="c", subcore_axis_name="s")
    kfn = pl.kernel(
        body,
        mesh=mesh,
        out_type=[jax.ShapeDtypeStruct((NC, npad, width), jnp.float32)],
        scratch_types=[
            pltpu.VMEM((nchunk, CHUNK), jnp.int32),
            pltpu.VMEM((nchunk, CHUNK), jnp.int32),
            pltpu.VMEM((CHUNK, width), jnp.float32),
            pltpu.VMEM((CHUNK, width), jnp.float32),
            pltpu.VMEM((ZR, width), jnp.float32),
            pltpu.VMEM_SHARED((npad, width), jnp.float32),
            pltpu.SemaphoreType.DMA,
            pltpu.SemaphoreType.DMA,
        ],
        compiler_params=pltpu.CompilerParams(use_tc_tiling_on_sc=False),
    )
    return kfn(ei4, g)[0]


def _tc_layer0(p2, xp, degE, degO, W0L, W0R, W1):
    m = p2.shape[1]
    half = W0L.shape[1]
    hid = W0L.shape[0]
    nz = W1.shape[0]
    R = m // 2

    def body(p_ref, x_ref, de_ref, do_ref, w0l_ref, w0r_ref, w1_ref, o_ref):
        invE = 1.0 / (de_ref[...] + 1.0)
        invO = 1.0 / (do_ref[...] + 1.0)
        rEL = p_ref[0][:, :half] + x_ref[:, :half]
        rER = p_ref[1][:, :half] + x_ref[:, half:2 * half]
        tE = lax.dot_general(rEL, w0l_ref[...], (((1,), (1,)), ((), ())),
                             preferred_element_type=jnp.float32)
        tE = tE + lax.dot_general(rER, w0r_ref[...], (((1,), (1,)), ((), ())),
                                  preferred_element_type=jnp.float32)
        hE = jnp.maximum(invE * tE, 0.0)
        o_ref[:, :nz] = lax.dot_general(hE, w1_ref[...],
                                        (((1,), (1,)), ((), ())),
                                        preferred_element_type=jnp.float32)
        rOL = p_ref[0][:, half:] + x_ref[:, 2 * half:3 * half]
        rOR = p_ref[1][:, half:] + x_ref[:, 3 * half:]
        tO = lax.dot_general(rOL, w0l_ref[...], (((1,), (1,)), ((), ())),
                             preferred_element_type=jnp.float32)
        tO = tO + lax.dot_general(rOR, w0r_ref[...], (((1,), (1,)), ((), ())),
                                  preferred_element_type=jnp.float32)
        hO = jnp.maximum(invO * tO, 0.0)
        o_ref[:, nz:] = lax.dot_general(hO, w1_ref[...],
                                        (((1,), (1,)), ((), ())),
                                        preferred_element_type=jnp.float32)

    return pl.pallas_call(
        body,
        grid=(m // R,),
        in_specs=[
            pl.BlockSpec((2, R, 2 * half), lambda i: (0, i, 0)),
            pl.BlockSpec((R, 4 * half), lambda i: (i, 0)),
            pl.BlockSpec((R, 1), lambda i: (i, 0)),
            pl.BlockSpec((R, 1), lambda i: (i, 0)),
            pl.BlockSpec((hid, half), lambda i: (0, 0)),
            pl.BlockSpec((hid, half), lambda i: (0, 0)),
            pl.BlockSpec((nz, hid), lambda i: (0, 0)),
        ],
        out_specs=pl.BlockSpec((R, 2 * nz), lambda i: (i, 0)),
        out_shape=jax.ShapeDtypeStruct((m, 2 * nz), jnp.float32),
    )(p2, xp, degE, degO, W0L, W0R, W1)


def _tc_layer1(q2, gp, degE, degO):
    m, w = gp.shape
    nz = w // 2
    R = m // 2

    def body(q_ref, g_ref, de_ref, do_ref, o_ref):
        invE = 1.0 / (de_ref[...] + 1.0)
        invO = 1.0 / (do_ref[...] + 1.0)
        s = q_ref[0] + q_ref[1] + g_ref[...]
        o_ref[:, :nz] = jnp.maximum(s[:, :nz] * invE, 0.0)
        o_ref[:, nz:] = jnp.maximum(s[:, nz:] * invO, 0.0)

    return pl.pallas_call(
        body,
        grid=(m // R,),
        in_specs=[
            pl.BlockSpec((2, R, w), lambda i: (0, i, 0)),
            pl.BlockSpec((R, w), lambda i: (i, 0)),
            pl.BlockSpec((R, 1), lambda i: (i, 0)),
            pl.BlockSpec((R, 1), lambda i: (i, 0)),
        ],
        out_specs=pl.BlockSpec((R, w), lambda i: (i, 0)),
        out_shape=jax.ShapeDtypeStruct((m, w), jnp.float32),
    )(q2, gp, degE, degO)


def kernel(x, edge_index, W0, W1):
    n, d_in = x.shape
    half = d_in // 2
    e = edge_index.shape[1]
    npad = ((n + 1023) // 1024) * 1024
    m = npad // 2

    ei4 = edge_index.astype(jnp.int32).reshape(2, NW, e // (NW * CHUNK), CHUNK)

    x2 = x.reshape(2 * n, half)
    xp = jnp.pad(x, ((0, npad - n), (0, 0))).reshape(m, 2 * d_in)

    p, deg2 = _sc_layer0(x2, ei4, npad, half)
    degE = deg2[0, 0:npad:2, None]
    degO = deg2[0, 1:npad:2, None]

    p2 = p.reshape(2, m, 2 * half)
    gp = _tc_layer0(p2, xp, degE, degO, W0[:, :half], W0[:, half:], W1)

    g = gp.reshape(npad, half)
    q = _sc_layer1(g, ei4, npad)

    out = _tc_layer1(q.reshape(2, m, 2 * half), gp, degE, degO)
    return out.reshape(npad, half)[:n]
